# baseline (speedup 1.0000x reference)
import functools

import jax
import jax.numpy as jnp
from jax.experimental import pallas as pl
from jax.experimental.pallas import tpu as pltpu

_R = 64
_K = 8


def _body(x1_ref, x2_ref, at_ref, bt_ref, d_ref, out_ref):
    x1 = x1_ref[...].astype(jnp.bfloat16)
    x2 = x2_ref[...].astype(jnp.bfloat16)
    h = at_ref.shape[0] // 2
    dn = (((1,), (0,)), ((), ()))
    y = jax.lax.dot_general(
        x1, at_ref[:h, :], dn, preferred_element_type=jnp.float32)
    y = y + jax.lax.dot_general(
        x2, at_ref[h:, :], dn, preferred_element_type=jnp.float32)
    yb = y + d_ref[...]
    a = jnp.abs(yb)

    idx = jax.lax.broadcasted_iota(jnp.int32, a.shape, 1)
    mask = jnp.zeros(a.shape, dtype=jnp.bool_)
    work = a
    for _ in range(_K):
        am = jnp.argmax(work, axis=1)
        sel = idx == am[:, None]
        mask = jnp.logical_or(mask, sel)
        work = jnp.where(sel, -jnp.inf, work)

    act = jnp.where(mask, y + y, 0.0).astype(jnp.bfloat16)
    out_ref[...] = jax.lax.dot_general(
        act, bt_ref[...], dn, preferred_element_type=jnp.float32)


@jax.jit
def kernel(x, A, B, d):
    n, in_f = x.shape
    out_f = B.shape[0]
    bt = 512
    h = in_f // 2
    grid = (n // bt,)
    return pl.pallas_call(
        _body,
        grid=grid,
        in_specs=[
            pl.BlockSpec((bt, h), lambda i: (i, 0)),
            pl.BlockSpec((bt, h), lambda i: (i, 1)),
            pl.BlockSpec((in_f, _R), lambda i: (0, 0)),
            pl.BlockSpec((_R, out_f), lambda i: (0, 0)),
            pl.BlockSpec((1, _R), lambda i: (0, 0)),
        ],
        out_specs=pl.BlockSpec((bt, out_f), lambda i: (i, 0)),
        out_shape=jax.ShapeDtypeStruct((n, out_f), jnp.float32),
        compiler_params=pltpu.CompilerParams(
            dimension_semantics=("parallel",)),
    )(x, x, A.T.astype(jnp.bfloat16), B.T.astype(jnp.bfloat16),
      d.reshape(1, _R))


# speedup vs baseline: 1.0068x; 1.0068x over previous
import jax
import jax.numpy as jnp
from jax.experimental import pallas as pl
from jax.experimental.pallas import tpu as pltpu

_R = 64
_K = 8


def _topk_mask(a):
    idx = jax.lax.broadcasted_iota(jnp.int32, a.shape, 1)
    mask = jnp.zeros(a.shape, dtype=jnp.bool_)
    work = a
    for _ in range(_K):
        am = jnp.argmax(work, axis=1)
        sel = idx == am[:, None]
        mask = jnp.logical_or(mask, sel)
        work = jnp.where(sel, -jnp.inf, work)
    return mask


def _body(x_ref, at_ref, bt_ref, d_ref, out_ref):
    x = x_ref[...].astype(jnp.bfloat16)
    dn = (((1,), (0,)), ((), ()))
    y = jax.lax.dot_general(
        x, at_ref[...], dn, preferred_element_type=jnp.float32)
    bt = x.shape[0]
    h = bt // 2
    for s in range(2):
        ys = y[s * h:(s + 1) * h, :]
        a = jnp.abs(ys + d_ref[...])
        mask = _topk_mask(a)
        act = jnp.where(mask, ys + ys, 0.0).astype(jnp.bfloat16)
        out_ref[s * h:(s + 1) * h, :] = jax.lax.dot_general(
            act, bt_ref[...], dn, preferred_element_type=jnp.float32)


@jax.jit
def kernel(x, A, B, d):
    n, in_f = x.shape
    out_f = B.shape[0]
    bt = 512
    grid = (n // bt,)
    return pl.pallas_call(
        _body,
        grid=grid,
        in_specs=[
            pl.BlockSpec((bt, in_f), lambda i: (i, 0)),
            pl.BlockSpec((in_f, _R), lambda i: (0, 0)),
            pl.BlockSpec((_R, out_f), lambda i: (0, 0)),
            pl.BlockSpec((1, _R), lambda i: (0, 0)),
        ],
        out_specs=pl.BlockSpec((bt, out_f), lambda i: (i, 0)),
        out_shape=jax.ShapeDtypeStruct((n, out_f), jnp.float32),
        compiler_params=pltpu.CompilerParams(
            dimension_semantics=("parallel",)),
    )(x, A.T.astype(jnp.bfloat16), B.T.astype(jnp.bfloat16), d.reshape(1, _R))
